# baseline (speedup 1.0000x reference)
import functools
import math

import jax
import jax.numpy as jnp
from jax import lax
from jax.experimental import pallas as pl
from jax.experimental.pallas import tpu as pltpu
from jax.experimental.pallas import tpu_sc as plsc

_M = 0.5
_COS_M = math.cos(_M)
_SIN_M = math.sin(_M)
_TH = math.cos(math.pi - _M)
_MM = math.sin(math.pi - _M) * _M
_EPS = 1e-07

_R = 1024
_C = 100000
_BC = 1024

_NC = 2
_NS = 16
_L = 16
_NW = _NC * _NS
_RPW = _R // _NW

def _sc_phi_body(cos_hbm, lbl_hbm, out_hbm, lbl_v, idx_v, val_v, phi_v, sem):
    wid = lax.axis_index("s") * _NC + lax.axis_index("c")
    base = wid * _RPW
    pltpu.sync_copy(lbl_hbm.at[pl.ds(base, _RPW)], lbl_v)
    for k in range(_RPW // _L):
        lbl = lbl_v[pl.ds(k * _L, _L)]
        rows = lax.iota(jnp.int32, _L) + (base + k * _L)
        idx_v[pl.ds(k * _L, _L)] = rows * _C + lbl
    pltpu.async_copy(cos_hbm.at[idx_v], val_v, sem).wait()
    for k in range(_RPW // _L):
        c = val_v[pl.ds(k * _L, _L)]
        c = jnp.minimum(jnp.maximum(c, -1.0 + _EPS), 1.0 - _EPS)
        x = 1.0 - c * c
        i = lax.bitcast_convert_type(x, jnp.int32)
        y = lax.bitcast_convert_type(
            lax.shift_right_arithmetic(i, 1) + 0x1FBD1DF5, jnp.float32)
        y = 0.5 * (y + x / y)
        y = 0.5 * (y + x / y)
        y = 0.5 * (y + x / y)
        phi = c * _COS_M - y * _SIN_M
        phi = jnp.where(c > _TH, phi, c - _MM)
        phi_v[pl.ds(k * _L, _L)] = phi
    pltpu.sync_copy(phi_v, out_hbm.at[pl.ds(base, _RPW)])


@functools.cache
def _sc_phi():
    mesh = plsc.VectorSubcoreMesh(core_axis_name="c", subcore_axis_name="s")
    return pl.kernel(
        _sc_phi_body,
        out_type=jax.ShapeDtypeStruct((_R,), jnp.float32),
        mesh=mesh,
        scratch_types=[
            pltpu.VMEM((_RPW,), jnp.int32),
            pltpu.VMEM((_RPW,), jnp.int32),
            pltpu.VMEM((_RPW,), jnp.float32),
            pltpu.VMEM((_RPW,), jnp.float32),
            pltpu.SemaphoreType.DMA,
        ],
    )


def _tc_body(s_ref, lbl_ref, phi_ref, x_ref, o_ref):
    j = pl.program_id(0)
    cols = lax.broadcasted_iota(jnp.int32, (_R, _BC), 1) + j * _BC
    mask = cols == lbl_ref[...]
    s = s_ref[0]
    o_ref[...] = jnp.where(mask, phi_ref[...], x_ref[...]) * s


def kernel(cosine, labels, s):
    lbl = labels.astype(jnp.int32)
    phi = _sc_phi()(jnp.reshape(cosine, (-1,)), lbl)
    s_arr = jnp.reshape(jnp.asarray(s, jnp.float32), (1,))
    lbl2 = jnp.reshape(lbl, (_R, 1))
    phi2 = jnp.reshape(phi, (_R, 1))
    return pl.pallas_call(
        _tc_body,
        grid=(pl.cdiv(_C, _BC),),
        in_specs=[
            pl.BlockSpec(memory_space=pltpu.SMEM),
            pl.BlockSpec((_R, 1), lambda j: (0, 0)),
            pl.BlockSpec((_R, 1), lambda j: (0, 0)),
            pl.BlockSpec((_R, _BC), lambda j: (0, j)),
        ],
        out_specs=pl.BlockSpec((_R, _BC), lambda j: (0, j)),
        out_shape=jax.ShapeDtypeStruct((_R, _C), jnp.float32),
        compiler_params=pltpu.CompilerParams(
            dimension_semantics=("arbitrary",),
        ),
    )(s_arr, lbl2, phi2, cosine)


# speedup vs baseline: 1.0050x; 1.0050x over previous
import functools
import math

import jax
import jax.numpy as jnp
from jax import lax
from jax.experimental import pallas as pl
from jax.experimental.pallas import tpu as pltpu
from jax.experimental.pallas import tpu_sc as plsc

_M = 0.5
_COS_M = math.cos(_M)
_SIN_M = math.sin(_M)
_TH = math.cos(math.pi - _M)
_MM = math.sin(math.pi - _M) * _M
_EPS = 1e-07

_R = 1024
_C = 100000
_BR = 16

_NC = 2
_NS = 16
_L = 16
_NW = _NC * _NS
_RPW = _R // _NW

def _sc_phi_body(cos_hbm, lbl_hbm, out_hbm, lbl_v, idx_v, val_v, phi_v, sem):
    wid = lax.axis_index("s") * _NC + lax.axis_index("c")
    base = wid * _RPW
    pltpu.sync_copy(lbl_hbm.at[pl.ds(base, _RPW)], lbl_v)
    for k in range(_RPW // _L):
        lbl = lbl_v[pl.ds(k * _L, _L)]
        rows = lax.iota(jnp.int32, _L) + (base + k * _L)
        idx_v[pl.ds(k * _L, _L)] = rows * _C + lbl
    pltpu.async_copy(cos_hbm.at[idx_v], val_v, sem).wait()
    for k in range(_RPW // _L):
        c = val_v[pl.ds(k * _L, _L)]
        c = jnp.minimum(jnp.maximum(c, -1.0 + _EPS), 1.0 - _EPS)
        x = 1.0 - c * c
        i = lax.bitcast_convert_type(x, jnp.int32)
        y = lax.bitcast_convert_type(
            lax.shift_right_arithmetic(i, 1) + 0x1FBD1DF5, jnp.float32)
        y = 0.5 * (y + x / y)
        y = 0.5 * (y + x / y)
        y = 0.5 * (y + x / y)
        phi = c * _COS_M - y * _SIN_M
        phi = jnp.where(c > _TH, phi, c - _MM)
        phi_v[pl.ds(k * _L, _L)] = phi
    pltpu.sync_copy(phi_v, out_hbm.at[pl.ds(base, _RPW)])


@functools.cache
def _sc_phi():
    mesh = plsc.VectorSubcoreMesh(core_axis_name="c", subcore_axis_name="s")
    return pl.kernel(
        _sc_phi_body,
        out_type=jax.ShapeDtypeStruct((_R,), jnp.float32),
        mesh=mesh,
        scratch_types=[
            pltpu.VMEM((_RPW,), jnp.int32),
            pltpu.VMEM((_RPW,), jnp.int32),
            pltpu.VMEM((_RPW,), jnp.float32),
            pltpu.VMEM((_RPW,), jnp.float32),
            pltpu.SemaphoreType.DMA,
        ],
    )


def _tc_body(s_ref, lbl_ref, phi_ref, x_ref, o_ref):
    cols = lax.broadcasted_iota(jnp.int32, (_BR, _C), 1)
    mask = cols == lbl_ref[...]
    s = s_ref[0]
    o_ref[...] = jnp.where(mask, phi_ref[...], x_ref[...]) * s


def kernel(cosine, labels, s):
    lbl = labels.astype(jnp.int32)
    phi = _sc_phi()(jnp.reshape(cosine, (-1,)), lbl)
    s_arr = jnp.reshape(jnp.asarray(s, jnp.float32), (1,))
    lbl2 = jnp.reshape(lbl, (_R, 1))
    phi2 = jnp.reshape(phi, (_R, 1))
    return pl.pallas_call(
        _tc_body,
        grid=(_R // _BR,),
        in_specs=[
            pl.BlockSpec(memory_space=pltpu.SMEM),
            pl.BlockSpec((_BR, 1), lambda i: (i, 0)),
            pl.BlockSpec((_BR, 1), lambda i: (i, 0)),
            pl.BlockSpec((_BR, _C), lambda i: (i, 0)),
        ],
        out_specs=pl.BlockSpec((_BR, _C), lambda i: (i, 0)),
        out_shape=jax.ShapeDtypeStruct((_R, _C), jnp.float32),
        compiler_params=pltpu.CompilerParams(
            dimension_semantics=("arbitrary",),
        ),
    )(s_arr, lbl2, phi2, cosine)


# speedup vs baseline: 1.6113x; 1.6032x over previous
import functools
import math

import jax
import jax.numpy as jnp
from jax import lax
from jax.experimental import pallas as pl
from jax.experimental.pallas import tpu as pltpu
from jax.experimental.pallas import tpu_sc as plsc

_M = 0.5
_COS_M = math.cos(_M)
_SIN_M = math.sin(_M)
_TH = math.cos(math.pi - _M)
_MM = math.sin(math.pi - _M) * _M
_EPS = 1e-07

_R = 1024
_C = 100000
_BR = 16

_NC = 2
_NS = 16
_L = 16
_NW = _NC * _NS
_RPW = _R // _NW

def _sc_phi_body(cos_hbm, lbl_hbm, out_hbm, lbl_v, idx_v, val_v, phi_v, sem):
    wid = lax.axis_index("s") * _NC + lax.axis_index("c")
    base = wid * _RPW
    pltpu.sync_copy(lbl_hbm.at[pl.ds(base, _RPW)], lbl_v)
    for k in range(_RPW // _L):
        lbl = lbl_v[pl.ds(k * _L, _L)]
        rows = lax.iota(jnp.int32, _L) + (base + k * _L)
        idx_v[pl.ds(k * _L, _L)] = rows * _C + lbl
    pltpu.async_copy(cos_hbm.at[idx_v], val_v, sem).wait()
    for k in range(_RPW // _L):
        c = val_v[pl.ds(k * _L, _L)]
        c = jnp.minimum(jnp.maximum(c, -1.0 + _EPS), 1.0 - _EPS)
        x = 1.0 - c * c
        i = lax.bitcast_convert_type(x, jnp.int32)
        y = lax.bitcast_convert_type(
            lax.shift_right_arithmetic(i, 1) + 0x1FBD1DF5, jnp.float32)
        y = 0.5 * (y + x / y)
        y = 0.5 * (y + x / y)
        y = 0.5 * (y + x / y)
        phi = c * _COS_M - y * _SIN_M
        phi = jnp.where(c > _TH, phi, c - _MM)
        phi_v[pl.ds(k * _L, _L)] = phi
    pltpu.sync_copy(phi_v, out_hbm.at[pl.ds(base, _RPW)])


@functools.cache
def _sc_phi():
    mesh = plsc.VectorSubcoreMesh(core_axis_name="c", subcore_axis_name="s")
    return pl.kernel(
        _sc_phi_body,
        out_type=jax.ShapeDtypeStruct((_R,), jnp.float32),
        mesh=mesh,
        scratch_types=[
            pltpu.VMEM((_RPW,), jnp.int32),
            pltpu.VMEM((_RPW,), jnp.int32),
            pltpu.VMEM((_RPW,), jnp.float32),
            pltpu.VMEM((_RPW,), jnp.float32),
            pltpu.SemaphoreType.DMA,
        ],
    )


def _tc_body(s_ref, lbl_ref, x_ref, o_ref):
    cols = lax.broadcasted_iota(jnp.int32, (_BR, _C), 1)
    mask = cols == lbl_ref[...]
    x = x_ref[...]
    ct = jnp.sum(jnp.where(mask, x, 0.0), axis=1, keepdims=True)
    ct = jnp.minimum(jnp.maximum(ct, -1.0 + _EPS), 1.0 - _EPS)
    sine = jnp.sqrt(1.0 - ct * ct)
    phi = ct * _COS_M - sine * _SIN_M
    phi = jnp.where(ct > _TH, phi, ct - _MM)
    o_ref[...] = jnp.where(mask, phi, x) * s_ref[0]


def kernel(cosine, labels, s):
    lbl = labels.astype(jnp.int32)
    s_arr = jnp.reshape(jnp.asarray(s, jnp.float32), (1,))
    lbl2 = jnp.reshape(lbl, (_R, 1))
    return pl.pallas_call(
        _tc_body,
        grid=(_R // _BR,),
        in_specs=[
            pl.BlockSpec(memory_space=pltpu.SMEM),
            pl.BlockSpec((_BR, 1), lambda i: (i, 0)),
            pl.BlockSpec((_BR, _C), lambda i: (i, 0)),
        ],
        out_specs=pl.BlockSpec((_BR, _C), lambda i: (i, 0)),
        out_shape=jax.ShapeDtypeStruct((_R, _C), jnp.float32),
        compiler_params=pltpu.CompilerParams(
            dimension_semantics=("arbitrary",),
        ),
    )(s_arr, lbl2, cosine)
